# baseline (speedup 1.0000x reference)
import functools

import jax
import jax.numpy as jnp
import numpy as np
from jax import lax
from jax.experimental import pallas as pl
from jax.experimental.pallas import tpu as pltpu
from jax.experimental.pallas import tpu_sc as plsc

NC = 2
NS = 16
NW = NC * NS

CH = 40


def _gather_body(EW, NCH, hid_hbm, rel_hbm, qemb_hbm, si_hbm, ri_hbm, qi_hbm,
                 hs_hbm, hr_hbm, hqr_hbm, idx_v, rows_v, sem):
    wid = lax.axis_index("s") * NC + lax.axis_index("c")
    pltpu.sync_copy(si_hbm.at[wid], idx_v.at[0])
    pltpu.sync_copy(ri_hbm.at[wid], idx_v.at[1])
    pltpu.sync_copy(qi_hbm.at[wid], idx_v.at[2])
    tabs = (hid_hbm, rel_hbm, qemb_hbm)
    outs = (hs_hbm, hr_hbm, hqr_hbm)

    def fire(ch, b):
        for t in range(3):
            pltpu.async_copy(tabs[t].at[idx_v.at[t, ch]], rows_v.at[b, t], sem)

    def drain(b):
        for t in range(3):
            pltpu.make_async_copy(tabs[t].at[idx_v.at[t, 0]],
                                  rows_v.at[b, t], sem).wait()

    def store(ch, b):
        base = wid * EW + ch * CH
        for t in range(3):
            pltpu.sync_copy(rows_v.at[b, t], outs[t].at[pl.ds(base, CH)])

    fire(0, 0)

    @pl.loop(0, (NCH - 1) // 2)
    def _pair(k):
        drain(0)
        fire(2 * k + 1, 1)
        store(2 * k, 0)
        drain(1)
        fire(2 * k + 2, 0)
        store(2 * k + 1, 1)

    drain(0)
    store(NCH - 1, 0)


def _scatter_body(NPAD, EW, NCHS, CHS, NSTR, msg_hbm, oi_hbm, zeros_hbm, part_hbm,
                  idx_v, rows_v, buf_v, acc_sh, sem):
    c = lax.axis_index("c")
    s = lax.axis_index("s")
    wid = s * NC + c
    stripe = NPAD // NS

    pltpu.sync_copy(zeros_hbm, buf_v)
    for j in range(NSTR):
        pltpu.sync_copy(buf_v, acc_sh.at[pl.ds(s * stripe + j * (stripe // NSTR),
                                               stripe // NSTR)])
    plsc.subcore_barrier()

    pltpu.sync_copy(oi_hbm.at[wid], idx_v)

    def fire(ch, b):
        base = wid * EW + ch * CHS
        pltpu.async_copy(msg_hbm.at[pl.ds(base, CHS)], rows_v.at[b], sem)

    def drain(b):
        pltpu.make_async_copy(msg_hbm.at[pl.ds(0, CHS)], rows_v.at[b], sem).wait()

    fire(0, 0)

    @pl.loop(0, (NCHS - 1) // 2)
    def _pair(k):
        drain(0)
        fire(2 * k + 1, 1)
        pltpu.sync_copy(rows_v.at[0], acc_sh.at[idx_v.at[2 * k]], add=True)
        drain(1)
        fire(2 * k + 2, 0)
        pltpu.sync_copy(rows_v.at[1], acc_sh.at[idx_v.at[2 * k + 1]], add=True)

    drain(0)
    pltpu.sync_copy(rows_v.at[0], acc_sh.at[idx_v.at[NCHS - 1]], add=True)

    plsc.subcore_barrier()

    for j in range(NSTR):
        r0 = s * stripe + j * (stripe // NSTR)
        pltpu.sync_copy(acc_sh.at[pl.ds(r0, stripe // NSTR)], buf_v)
        pltpu.sync_copy(buf_v, part_hbm.at[c, pl.ds(r0, stripe // NSTR)])


def _attn_body(hs_ref, hr_ref, hqr_ref, w1_ref, w2_ref, w3_ref, w4_ref,
               wsb_ref, wa_ref, wab_ref, one_ref, msg_ref, alpha_ref):
    hs = hs_ref[...]
    hr = hr_ref[...]
    hqr = hqr_ref[...]
    pre = (jnp.dot(hs, w1_ref[...], preferred_element_type=jnp.float32)
           + jnp.dot(hr, w2_ref[...], preferred_element_type=jnp.float32)
           + jnp.dot(hqr, w3_ref[...], preferred_element_type=jnp.float32)
           + jnp.dot(hr * hqr, w4_ref[...], preferred_element_type=jnp.float32))
    pre = jnp.maximum(pre + wsb_ref[...], 0.0)
    score_t = lax.dot_general(wa_ref[...], pre, (((1,), (1,)), ((), ())),
                              preferred_element_type=jnp.float32)
    alpha_t = jax.nn.sigmoid(score_t + wab_ref[...])
    alpha_col = lax.dot_general(alpha_t, one_ref[...], (((0,), (0,)), ((), ())),
                                preferred_element_type=jnp.float32)
    msg_ref[...] = alpha_col * (hs * hr)
    alpha_ref[...] = alpha_t.reshape(alpha_ref.shape)


def _final_body(p_ref, wht_ref, out_ref):
    agg = (p_ref[0, 0] + p_ref[0, 1]) + (p_ref[1, 0] + p_ref[1, 1])
    out_ref[...] = jnp.dot(agg, wht_ref[...],
                           preferred_element_type=jnp.float32)


def kernel(q_sub, q_rel, q_emb, rela_embed, hidden, edges, nodes,
           old_nodes_new_idx, Ws_w, Ws_b, Wr_w, Wq_w, Wqr_w, wa_w, wa_b, Wh_w):
    N, D = hidden.shape
    E = edges.shape[0]
    A = Ws_w.shape[0]
    NH = 2
    EH = E // NH
    EW = EH // NW
    NCH = EW // CH
    CHS = CH
    NCHS = NCH
    NSTR = 16
    assert EH * NH == E and EW * NW == EH and NCH * CH == EW and NCH % 2 == 1

    sub = edges[:, 4].astype(jnp.int32).reshape(NH, NW, NCH, CH)
    rel = edges[:, 2].astype(jnp.int32).reshape(NH, NW, NCH, CH)
    qid = edges[:, 0].astype(jnp.int32).reshape(NH, NW, NCH, CH)
    obj = edges[:, 5].astype(jnp.int32).reshape(NH, NW, NCHS, CHS)

    mesh = plsc.VectorSubcoreMesh(core_axis_name="c", subcore_axis_name="s")
    gather = pl.kernel(
        functools.partial(_gather_body, EW, NCH),
        out_type=[jax.ShapeDtypeStruct((EH, D), jnp.float32)] * 3,
        mesh=mesh,
        scratch_types=[
            pltpu.VMEM((3, NCH, CH), jnp.int32),
            pltpu.VMEM((2, 3, CH, D), jnp.float32),
            pltpu.SemaphoreType.DMA,
        ],
    )

    wsb = Ws_b.reshape(1, A)
    wab = wa_b.reshape(1, 1)
    BLK = 6400
    nblk = EH // BLK
    attn = pl.pallas_call(
        _attn_body,
        grid=(nblk,),
        in_specs=[
            pl.BlockSpec((BLK, D), lambda i: (i, 0)),
            pl.BlockSpec((BLK, D), lambda i: (i, 0)),
            pl.BlockSpec((BLK, D), lambda i: (i, 0)),
            pl.BlockSpec((D, A), lambda i: (0, 0)),
            pl.BlockSpec((D, A), lambda i: (0, 0)),
            pl.BlockSpec((D, A), lambda i: (0, 0)),
            pl.BlockSpec((D, A), lambda i: (0, 0)),
            pl.BlockSpec((1, A), lambda i: (0, 0)),
            pl.BlockSpec((1, A), lambda i: (0, 0)),
            pl.BlockSpec((1, 1), lambda i: (0, 0)),
            pl.BlockSpec((1, 1), lambda i: (0, 0)),
        ],
        out_specs=[
            pl.BlockSpec((BLK, D), lambda i: (i, 0)),
            pl.BlockSpec((1, 1, BLK), lambda i: (i, 0, 0)),
        ],
        out_shape=[
            jax.ShapeDtypeStruct((EH, D), jnp.float32),
            jax.ShapeDtypeStruct((nblk, 1, BLK), jnp.float32),
        ],
    )

    NPAD = 10240
    assert NPAD >= N and NPAD % (NS * NSTR * 8) == 0
    zeros = jnp.zeros((NPAD // NS // NSTR, D), jnp.float32)
    scatter = pl.kernel(
        functools.partial(_scatter_body, NPAD, EW, NCHS, CHS, NSTR),
        out_type=jax.ShapeDtypeStruct((NC, NPAD, D), jnp.float32),
        mesh=mesh,
        scratch_types=[
            pltpu.VMEM((NCHS, CHS), jnp.int32),
            pltpu.VMEM((2, CHS, D), jnp.float32),
            pltpu.VMEM((NPAD // NS // NSTR, D), jnp.float32),
            pltpu.VMEM_SHARED((NPAD, D), jnp.float32),
            pltpu.SemaphoreType.DMA,
        ],
    )

    ones11 = jnp.ones((1, 1), jnp.float32)
    alphas, parts = [], []
    for h in range(NH):
        hs, hr, hqr = gather(hidden, rela_embed, q_emb, sub[h], rel[h], qid[h])
        msg, alpha_h = attn(hs, hr, hqr, Ws_w.T, Wr_w.T, Wq_w.T, Wqr_w.T,
                            wsb, wa_w, wab, ones11)
        alphas.append(alpha_h.reshape(EH, 1))
        parts.append(scatter(msg, obj[h], zeros))
    alpha = jnp.concatenate(alphas, axis=0)

    hidden_new = pl.pallas_call(
        _final_body,
        grid=(1,),
        in_specs=[
            pl.BlockSpec((NH, NC, N, D), lambda i: (0, 0, 0, 0)),
            pl.BlockSpec((D, D), lambda i: (0, 0)),
        ],
        out_specs=pl.BlockSpec((N, D), lambda i: (0, 0)),
        out_shape=jax.ShapeDtypeStruct((N, D), jnp.float32),
    )(jnp.stack(parts), Wh_w.T)

    B = q_sub.shape[0]
    n1 = nodes.shape[0]
    num_node = np.array([n1 * 1.0 / B, n1 * 1.0 / B])
    num_edge = np.array([E * 1.0 / B, E * 1.0 / B])
    return (num_node, num_edge, hidden_new, alpha, nodes, edges,
            old_nodes_new_idx)


# speedup vs baseline: 1.0958x; 1.0958x over previous
import functools

import jax
import jax.numpy as jnp
import numpy as np
from jax import lax
from jax.experimental import pallas as pl
from jax.experimental.pallas import tpu as pltpu
from jax.experimental.pallas import tpu_sc as plsc

NC = 2
NS = 16
NW = NC * NS

CH = 80


def _gather_body(EW, NCH, hid_hbm, rel_hbm, qemb_hbm, si_hbm, ri_hbm, qi_hbm,
                 hs_hbm, hr_hbm, hqr_hbm, idx_v, rows_v, sem):
    wid = lax.axis_index("s") * NC + lax.axis_index("c")
    pltpu.sync_copy(si_hbm.at[wid], idx_v.at[0])
    pltpu.sync_copy(ri_hbm.at[wid], idx_v.at[1])
    pltpu.sync_copy(qi_hbm.at[wid], idx_v.at[2])
    tabs = (hid_hbm, rel_hbm, qemb_hbm)
    outs = (hs_hbm, hr_hbm, hqr_hbm)

    def fire(ch, b):
        for t in range(3):
            pltpu.async_copy(tabs[t].at[idx_v.at[t, ch]], rows_v.at[b, t], sem)

    def drain(b):
        for t in range(3):
            pltpu.make_async_copy(tabs[t].at[idx_v.at[t, 0]],
                                  rows_v.at[b, t], sem).wait()

    def store(ch, b):
        base = wid * EW + ch * CH
        for t in range(3):
            pltpu.sync_copy(rows_v.at[b, t], outs[t].at[pl.ds(base, CH)])

    fire(0, 0)

    @pl.loop(0, NCH // 2)
    def _pair(k):
        drain(0)
        fire(2 * k + 1, 1)
        store(2 * k, 0)
        drain(1)

        @pl.when(2 * k + 2 < NCH)
        def _():
            fire(2 * k + 2, 0)

        store(2 * k + 1, 1)

    if NCH % 2 == 1:
        drain(0)
        store(NCH - 1, 0)


def _scatter_body(NPAD, EW, NCHS, CHS, NSTR, msg_hbm, oi_hbm, zeros_hbm, part_hbm,
                  idx_v, rows_v, buf_v, acc_sh, sem):
    c = lax.axis_index("c")
    s = lax.axis_index("s")
    wid = s * NC + c
    stripe = NPAD // NS

    pltpu.sync_copy(zeros_hbm, buf_v)
    for j in range(NSTR):
        pltpu.sync_copy(buf_v, acc_sh.at[pl.ds(s * stripe + j * (stripe // NSTR),
                                               stripe // NSTR)])
    plsc.subcore_barrier()

    pltpu.sync_copy(oi_hbm.at[wid], idx_v)

    def fire(ch, b):
        base = wid * EW + ch * CHS
        pltpu.async_copy(msg_hbm.at[pl.ds(base, CHS)], rows_v.at[b], sem)

    def drain(b):
        pltpu.make_async_copy(msg_hbm.at[pl.ds(0, CHS)], rows_v.at[b], sem).wait()

    fire(0, 0)

    @pl.loop(0, NCHS // 2)
    def _pair(k):
        drain(0)
        fire(2 * k + 1, 1)
        pltpu.sync_copy(rows_v.at[0], acc_sh.at[idx_v.at[2 * k]], add=True)
        drain(1)

        @pl.when(2 * k + 2 < NCHS)
        def _():
            fire(2 * k + 2, 0)

        pltpu.sync_copy(rows_v.at[1], acc_sh.at[idx_v.at[2 * k + 1]], add=True)

    if NCHS % 2 == 1:
        drain(0)
        pltpu.sync_copy(rows_v.at[0], acc_sh.at[idx_v.at[NCHS - 1]], add=True)

    plsc.subcore_barrier()

    for j in range(NSTR):
        r0 = s * stripe + j * (stripe // NSTR)
        pltpu.sync_copy(acc_sh.at[pl.ds(r0, stripe // NSTR)], buf_v)
        pltpu.sync_copy(buf_v, part_hbm.at[c, pl.ds(r0, stripe // NSTR)])


def _attn_body(hs_ref, hr_ref, hqr_ref, w1_ref, w2_ref, w3_ref, w4_ref,
               wsb_ref, wa_ref, wab_ref, one_ref, msg_ref, alpha_ref):
    hs = hs_ref[...]
    hr = hr_ref[...]
    hqr = hqr_ref[...]
    pre = (jnp.dot(hs, w1_ref[...], preferred_element_type=jnp.float32)
           + jnp.dot(hr, w2_ref[...], preferred_element_type=jnp.float32)
           + jnp.dot(hqr, w3_ref[...], preferred_element_type=jnp.float32)
           + jnp.dot(hr * hqr, w4_ref[...], preferred_element_type=jnp.float32))
    pre = jnp.maximum(pre + wsb_ref[...], 0.0)
    score_t = lax.dot_general(wa_ref[...], pre, (((1,), (1,)), ((), ())),
                              preferred_element_type=jnp.float32)
    alpha_t = jax.nn.sigmoid(score_t + wab_ref[...])
    alpha_col = lax.dot_general(alpha_t, one_ref[...], (((0,), (0,)), ((), ())),
                                preferred_element_type=jnp.float32)
    msg_ref[...] = alpha_col * (hs * hr)
    alpha_ref[...] = alpha_t.reshape(alpha_ref.shape)


def _final_body(p_ref, wht_ref, out_ref):
    agg = (p_ref[0, 0] + p_ref[0, 1]) + (p_ref[1, 0] + p_ref[1, 1])
    out_ref[...] = jnp.dot(agg, wht_ref[...],
                           preferred_element_type=jnp.float32)


def kernel(q_sub, q_rel, q_emb, rela_embed, hidden, edges, nodes,
           old_nodes_new_idx, Ws_w, Ws_b, Wr_w, Wq_w, Wqr_w, wa_w, wa_b, Wh_w):
    N, D = hidden.shape
    E = edges.shape[0]
    A = Ws_w.shape[0]
    EHS = (166400, 153600)
    CHS = 40
    NSTR = 16
    BLK = 6400
    assert sum(EHS) == E

    sub_f = edges[:, 4].astype(jnp.int32)
    rel_f = edges[:, 2].astype(jnp.int32)
    qid_f = edges[:, 0].astype(jnp.int32)
    obj_f = edges[:, 5].astype(jnp.int32)

    mesh = plsc.VectorSubcoreMesh(core_axis_name="c", subcore_axis_name="s")
    wsb = Ws_b.reshape(1, A)
    wab = wa_b.reshape(1, 1)
    ones11 = jnp.ones((1, 1), jnp.float32)
    NPAD = 10240
    assert NPAD >= N and NPAD % (NS * NSTR * 8) == 0
    zeros = jnp.zeros((NPAD // NS // NSTR, D), jnp.float32)

    alphas, parts, e0 = [], [], 0
    for EH in EHS:
        EW = EH // NW
        NCH = EW // CH
        NCHS = EW // CHS
        assert NCH * CH == EW and NCHS * CHS == EW

        sub = lax.dynamic_slice_in_dim(sub_f, e0, EH).reshape(NW, NCH, CH)
        rel = lax.dynamic_slice_in_dim(rel_f, e0, EH).reshape(NW, NCH, CH)
        qid = lax.dynamic_slice_in_dim(qid_f, e0, EH).reshape(NW, NCH, CH)
        obj = lax.dynamic_slice_in_dim(obj_f, e0, EH).reshape(NW, NCHS, CHS)
        e0 += EH

        gather = pl.kernel(
            functools.partial(_gather_body, EW, NCH),
            out_type=[jax.ShapeDtypeStruct((EH, D), jnp.float32)] * 3,
            mesh=mesh,
            scratch_types=[
                pltpu.VMEM((3, NCH, CH), jnp.int32),
                pltpu.VMEM((2, 3, CH, D), jnp.float32),
                pltpu.SemaphoreType.DMA,
            ],
        )
        hs, hr, hqr = gather(hidden, rela_embed, q_emb, sub, rel, qid)

        nblk = EH // BLK
        msg, alpha_h = pl.pallas_call(
            _attn_body,
            grid=(nblk,),
            in_specs=[
                pl.BlockSpec((BLK, D), lambda i: (i, 0)),
                pl.BlockSpec((BLK, D), lambda i: (i, 0)),
                pl.BlockSpec((BLK, D), lambda i: (i, 0)),
                pl.BlockSpec((D, A), lambda i: (0, 0)),
                pl.BlockSpec((D, A), lambda i: (0, 0)),
                pl.BlockSpec((D, A), lambda i: (0, 0)),
                pl.BlockSpec((D, A), lambda i: (0, 0)),
                pl.BlockSpec((1, A), lambda i: (0, 0)),
                pl.BlockSpec((1, A), lambda i: (0, 0)),
                pl.BlockSpec((1, 1), lambda i: (0, 0)),
                pl.BlockSpec((1, 1), lambda i: (0, 0)),
            ],
            out_specs=[
                pl.BlockSpec((BLK, D), lambda i: (i, 0)),
                pl.BlockSpec((1, 1, BLK), lambda i: (i, 0, 0)),
            ],
            out_shape=[
                jax.ShapeDtypeStruct((EH, D), jnp.float32),
                jax.ShapeDtypeStruct((nblk, 1, BLK), jnp.float32),
            ],
        )(hs, hr, hqr, Ws_w.T, Wr_w.T, Wq_w.T, Wqr_w.T, wsb, wa_w, wab, ones11)
        alphas.append(alpha_h.reshape(EH, 1))

        scatter = pl.kernel(
            functools.partial(_scatter_body, NPAD, EW, NCHS, CHS, NSTR),
            out_type=jax.ShapeDtypeStruct((NC, NPAD, D), jnp.float32),
            mesh=mesh,
            scratch_types=[
                pltpu.VMEM((NCHS, CHS), jnp.int32),
                pltpu.VMEM((2, CHS, D), jnp.float32),
                pltpu.VMEM((NPAD // NS // NSTR, D), jnp.float32),
                pltpu.VMEM_SHARED((NPAD, D), jnp.float32),
                pltpu.SemaphoreType.DMA,
            ],
        )
        parts.append(scatter(msg, obj, zeros))
    alpha = jnp.concatenate(alphas, axis=0)

    hidden_new = pl.pallas_call(
        _final_body,
        grid=(1,),
        in_specs=[
            pl.BlockSpec((2, NC, N, D), lambda i: (0, 0, 0, 0)),
            pl.BlockSpec((D, D), lambda i: (0, 0)),
        ],
        out_specs=pl.BlockSpec((N, D), lambda i: (0, 0)),
        out_shape=jax.ShapeDtypeStruct((N, D), jnp.float32),
    )(jnp.stack(parts), Wh_w.T)

    B = q_sub.shape[0]
    n1 = nodes.shape[0]
    num_node = np.array([n1 * 1.0 / B, n1 * 1.0 / B])
    num_edge = np.array([E * 1.0 / B, E * 1.0 / B])
    return (num_node, num_edge, hidden_new, alpha, nodes, edges,
            old_nodes_new_idx)
